# Initial kernel scaffold; baseline (speedup 1.0000x reference)
import jax
import jax.numpy as jnp
from jax.experimental import pallas as pl


def kernel(point_seq):
    raise NotImplementedError("write your pallas kernel here")


# speedup vs baseline: 1.8569x; 1.8569x over previous
import functools

import jax
import jax.numpy as jnp
from jax.experimental import pallas as pl
from jax.experimental.pallas import tpu as pltpu

K = 16


def _chain_kernel(xrow_ref, q0_ref, dist_ref, idx_ref, qout_ref, *, t, n, nk):
    q = q0_ref[0]
    iota_full = jax.lax.broadcasted_iota(jnp.int32, (n, n), 1)
    iota_keep = jax.lax.broadcasted_iota(jnp.int32, (nk, n), 1)
    for i in range(t):
        db0 = xrow_ref[0, 3 * i + 0, :][None, :]
        db1 = xrow_ref[0, 3 * i + 1, :][None, :]
        db2 = xrow_ref[0, 3 * i + 2, :][None, :]
        t0 = q[:, 0:1] - db0
        t1 = q[:, 1:2] - db1
        t2 = q[:, 2:3] - db2
        d2 = t0 * t0 + t1 * t1 + t2 * t2

        dist_cols = []
        idx_cols = []
        m = jnp.min(d2, axis=1, keepdims=True)
        cand = jnp.where(d2 == m, iota_full, jnp.int32(2**30))
        sel = jnp.min(cand, axis=1, keepdims=True)
        onehot = iota_full == sel
        qn0 = jnp.sum(jnp.where(onehot, db0, 0.0), axis=1, keepdims=True)
        qn1 = jnp.sum(jnp.where(onehot, db1, 0.0), axis=1, keepdims=True)
        qn2 = jnp.sum(jnp.where(onehot, db2, 0.0), axis=1, keepdims=True)
        q = jnp.concatenate([qn0, qn1, qn2], axis=1)
        qout_ref[0, i] = q
        dist_cols.append(m[:nk])
        idx_cols.append(sel[:nk])
        dk = jnp.where(onehot[:nk], jnp.inf, d2[:nk])
        for _ in range(1, K):
            m = jnp.min(dk, axis=1, keepdims=True)
            cand = jnp.where(dk == m, iota_keep, jnp.int32(2**30))
            sel = jnp.min(cand, axis=1, keepdims=True)
            onehot = iota_keep == sel
            dk = jnp.where(onehot, jnp.inf, dk)
            dist_cols.append(m)
            idx_cols.append(sel)
        dist_ref[0, i] = jnp.concatenate(dist_cols, axis=1)
        idx_ref[0, i] = jnp.concatenate(idx_cols, axis=1)


def _run_chains(xrow, q0, t, n, nk):
    c = xrow.shape[0]
    grid = (c,)
    return pl.pallas_call(
        functools.partial(_chain_kernel, t=t, n=n, nk=nk),
        grid=grid,
        in_specs=[
            pl.BlockSpec((1, 3 * t, n), lambda i: (i, 0, 0)),
            pl.BlockSpec((1, n, 3), lambda i: (i, 0, 0)),
        ],
        out_specs=[
            pl.BlockSpec((1, t, nk, K), lambda i: (i, 0, 0, 0)),
            pl.BlockSpec((1, t, nk, K), lambda i: (i, 0, 0, 0)),
            pl.BlockSpec((1, t, n, 3), lambda i: (i, 0, 0, 0)),
        ],
        out_shape=[
            jax.ShapeDtypeStruct((c, t, nk, K), jnp.float32),
            jax.ShapeDtypeStruct((c, t, nk, K), jnp.int32),
            jax.ShapeDtypeStruct((c, t, n, 3), jnp.float32),
        ],
        compiler_params=pltpu.CompilerParams(
            dimension_semantics=("parallel",),
        ),
    )(xrow, q0)


def kernel(point_seq):
    b, t, n, d = point_seq.shape
    k = K
    nk = n // 2
    X = jnp.concatenate([point_seq, jnp.flip(point_seq, 1)], axis=0)
    rand_idxs = jax.random.permutation(jax.random.key(42), n)[: n // 2]
    keep_mask = jnp.zeros((n,), bool).at[rand_idxs].set(True)
    rest = jnp.where(~keep_mask, size=n - nk)[0]
    perm = jnp.concatenate([rand_idxs, rest])
    inv = jnp.zeros((n,), jnp.int32).at[perm].set(jnp.arange(n, dtype=jnp.int32))

    q0 = X[:, 0][:, perm]
    xrow = X.transpose(0, 1, 3, 2).reshape(2 * b, 3 * t, n)

    dist, idx, qout = _run_chains(xrow, q0, t, n, nk)

    tbl_prev = jnp.concatenate([X[:, :1], X[:, :-1]], axis=1)
    cidx = jnp.arange(2 * b).reshape(2 * b, 1, 1, 1)
    tidx = jnp.arange(t).reshape(1, t, 1, 1)
    pts = X[cidx, tidx, idx]
    feats = tbl_prev[cidx, tidx, idx]

    def asm(a):
        return jnp.concatenate([a[:b], jnp.flip(a[b:], axis=1)], axis=2)

    distances = asm(dist)
    idxs = asm(idx)
    patchlet_points = asm(pts)
    patchlet_feats = asm(feats)
    out_x = qout[:b][:, :, inv]
    patchlets = idxs
    return (patchlet_points, patchlet_feats, distances, idxs, patchlets, out_x)
